# baseline (speedup 1.0000x reference)
import functools

import jax
import jax.numpy as jnp
from jax import lax
from jax.experimental import pallas as pl
from jax.experimental.pallas import tpu as pltpu
from jax.experimental.pallas import tpu_sc as plsc

BATCH = 16384
HIST = 50
EMB = 32
NUM_ROWS = BATCH * HIST

NC = 2
NS = 16
NW = NC * NS
ROWS_PER_W = NUM_ROWS // NW
CH = 1280
N_BIG = ROWS_PER_W // CH
NBUF = 2
N_GRP = N_BIG // NBUF

_mesh = plsc.VectorSubcoreMesh(core_axis_name="c", subcore_axis_name="s")


@functools.partial(
    pl.kernel,
    mesh=_mesh,
    out_type=jax.ShapeDtypeStruct((NUM_ROWS, EMB), jnp.float32),
    scratch_types=(
        [pltpu.VMEM((ROWS_PER_W,), jnp.int32)]
        + [pltpu.VMEM((CH, EMB), jnp.float32) for _ in range(NBUF)]
        + [pltpu.SemaphoreType.DMA for _ in range(2 * NBUF)]
    ),
    compiler_params=pltpu.CompilerParams(use_tc_tiling_on_sc=False),
)
def _emb_gather(x_hbm, table_hbm, out_hbm, idx_v, *rest):
    rows = rest[:NBUF]
    gsem = rest[NBUF:2 * NBUF]
    ssem = rest[2 * NBUF:]
    wid = lax.axis_index("s") * NC + lax.axis_index("c")
    base = wid * ROWS_PER_W
    pltpu.sync_copy(x_hbm.at[wid], idx_v)

    def start_gather(j, b):
        pltpu.async_copy(table_hbm.at[idx_v.at[pl.ds(j * CH, CH)]], rows[b], gsem[b])

    def wait_gather(b):
        pltpu.make_async_copy(out_hbm.at[pl.ds(base, CH)], rows[b], gsem[b]).wait()

    def start_store(j, b):
        pltpu.async_copy(rows[b], out_hbm.at[pl.ds(base + j * CH, CH)], ssem[b])

    def wait_store(b):
        pltpu.make_async_copy(rows[b], out_hbm.at[pl.ds(base, CH)], ssem[b]).wait()

    for b in range(NBUF):
        start_gather(b, b)

    def body(g, carry):
        for b in range(NBUF):
            wait_gather(b)
            start_store(g * NBUF + b, b)
        for b in range(NBUF):
            wait_store(b)
            start_gather((g + 1) * NBUF + b, b)
        return carry

    lax.fori_loop(0, N_GRP - 1, body, 0)

    for b in range(NBUF):
        wait_gather(b)
        start_store((N_GRP - 1) * NBUF + b, b)
    for b in range(NBUF):
        wait_store(b)


def kernel(x, table):
    x_grp = x.reshape(NW, ROWS_PER_W)
    out = _emb_gather(x_grp, table)
    return out.reshape(BATCH, HIST, EMB)


# speedup vs baseline: 1.6246x; 1.6246x over previous
import functools

import jax
import jax.numpy as jnp
from jax import lax
from jax.experimental import pallas as pl
from jax.experimental.pallas import tpu as pltpu
from jax.experimental.pallas import tpu_sc as plsc

BATCH = 16384
HIST = 50
EMB = 32

NC = 2
NS = 16
NW = NC * NS
B_PER_W = BATCH // NW
CB = 8
N_CHUNK = B_PER_W // CB
NBUF = 4
N_GRP = N_CHUNK // NBUF

_mesh = plsc.VectorSubcoreMesh(core_axis_name="c", subcore_axis_name="s")


@functools.partial(
    pl.kernel,
    mesh=_mesh,
    out_type=jax.ShapeDtypeStruct((BATCH, HIST, EMB), jnp.float32),
    scratch_types=(
        [pltpu.VMEM((B_PER_W, HIST), jnp.int32)]
        + [pltpu.VMEM((CB, HIST, EMB), jnp.float32) for _ in range(NBUF)]
        + [pltpu.SemaphoreType.DMA for _ in range(2 * NBUF)]
    ),
    compiler_params=pltpu.CompilerParams(use_tc_tiling_on_sc=False),
)
def _emb_gather(x_hbm, table_hbm, out_hbm, idx_v, *rest):
    rows = rest[:NBUF]
    gsem = rest[NBUF:2 * NBUF]
    ssem = rest[2 * NBUF:]
    wid = lax.axis_index("s") * NC + lax.axis_index("c")
    base = wid * B_PER_W
    pltpu.sync_copy(x_hbm.at[pl.ds(base, B_PER_W)], idx_v)

    def start_gather(j, b):
        for k in range(CB):
            pltpu.async_copy(
                table_hbm.at[idx_v.at[j * CB + k]], rows[b].at[k], gsem[b]
            )

    def wait_gather(b):
        pltpu.make_async_copy(out_hbm.at[pl.ds(0, CB)], rows[b], gsem[b]).wait()

    def start_store(j, b):
        pltpu.async_copy(rows[b], out_hbm.at[pl.ds(base + j * CB, CB)], ssem[b])

    def wait_store(b):
        pltpu.make_async_copy(rows[b], out_hbm.at[pl.ds(0, CB)], ssem[b]).wait()

    for b in range(NBUF):
        start_gather(b, b)

    def body(g, carry):
        for b in range(NBUF):
            wait_gather(b)
            start_store(g * NBUF + b, b)
        for b in range(NBUF):
            wait_store(b)
            start_gather((g + 1) * NBUF + b, b)
        return carry

    lax.fori_loop(0, N_GRP - 1, body, 0)

    for b in range(NBUF):
        wait_gather(b)
        start_store((N_GRP - 1) * NBUF + b, b)
    for b in range(NBUF):
        wait_store(b)


def kernel(x, table):
    return _emb_gather(x, table)
